# baseline (speedup 1.0000x reference)
import functools

import jax
import jax.numpy as jnp
from jax import lax
from jax.experimental import pallas as pl
from jax.experimental.pallas import tpu as pltpu
from jax.experimental.pallas import tpu_sc as plsc

B, NCTX, NNEGS, V, D = 4096, 20, 5, 1000000, 64
NNEG = NNEGS * NCTX
NSAM = NCTX + NNEG
NG = 8
NW = 32
BPW = B // NW
L = 16
D2 = 2 * D
CB = 2
NCH = BPW // CB
CROWS = CB * NSAM


def _sc_sim_body(curi_h, curo_h, ctxi_h, negi_h, samu_h, win_h, wout_h,
                 out_h, curi_v, curo_v, ctxi_v, negi_v, samu_v, currow_v,
                 rows_v, sim_v, sem0, sem1):
    wid = lax.axis_index("s") * 2 + lax.axis_index("c")
    base = wid * BPW
    sems = (sem0, sem1)

    pltpu.sync_copy(curi_h.at[pl.ds(base, BPW)], curi_v)
    pltpu.sync_copy(curo_h.at[pl.ds(base, BPW)], curo_v)
    pltpu.sync_copy(ctxi_h.at[pl.ds(base * NCTX, BPW * NCTX)], ctxi_v)
    pltpu.sync_copy(negi_h.at[pl.ds(base * NNEG, BPW * NNEG)], negi_v)
    pltpu.sync_copy(samu_h.at[pl.ds(base, BPW)], samu_v)
    pltpu.async_copy(win_h.at[curi_v], currow_v, sems[0]).wait()

    lanes = lax.iota(jnp.int32, L)
    rowsel0, rowstep = [], []
    for g in range(NG):
        sj = lanes + g * L
        row0 = jnp.where(sj < NCTX, sj, CB * NCTX + (sj - NCTX))
        step = jnp.where(sj < NCTX, NCTX, NNEG)
        valid = sj < NSAM
        rowsel0.append(jnp.where(valid, row0, 0))
        rowstep.append(jnp.where(valid, step, 0))
    sign = [jnp.where(lanes + g * L < NCTX, 1.0, -1.0).astype(jnp.float32)
            for g in range(NG)]
    zero = jnp.zeros((L,), jnp.float32)

    def issue(c, buf):
        pltpu.async_copy(wout_h.at[ctxi_v.at[pl.ds(c * CB * NCTX, CB * NCTX)]],
                         rows_v.at[buf, pl.ds(0, CB * NCTX)], sems[buf])
        pltpu.async_copy(wout_h.at[negi_v.at[pl.ds(c * CB * NNEG, CB * NNEG)]],
                         rows_v.at[buf, pl.ds(CB * NCTX, CB * NNEG)],
                         sems[buf])

    issue(0, 0)

    def body(cc, carry):
        for par in range(2):
            c = cc * 2 + par

            @pl.when(c + 1 < NCH)
            def _():
                issue(c + 1, 1 - par)

            pltpu.make_async_copy(
                wout_h.at[pl.ds(0, CROWS)],
                rows_v.at[par, pl.ds(0, CROWS)], sems[par]).wait()

            rows_c = rows_v.at[par]

            def bbody(bi, bcarry):
                b = c * CB + bi
                bvec = jnp.full((L,), b, jnp.int32)
                curov = plsc.load_gather(curo_v, [bvec])
                sel = [rowsel0[g] + bi * rowstep[g] for g in range(NG)]
                soff = [(samu_v[b, pl.ds(g * L, L)] & 1) * D
                        for g in range(NG)]
                accs = [zero] * NG
                for d in range(D):
                    colrot = (lanes + d) & (D - 1)
                    crot = plsc.load_gather(currow_v, [bvec, curov + colrot])
                    for g in range(NG):
                        accs[g] = accs[g] + plsc.load_gather(
                            rows_c, [sel[g], soff[g] + colrot]) * crot
                for g in range(NG):
                    sim_v[b, pl.ds(g * L, L)] = accs[g] * sign[g]
                return bcarry

            lax.fori_loop(0, CB, bbody, 0)
        return carry

    lax.fori_loop(0, NCH // 2, body, 0)
    pltpu.sync_copy(sim_v, out_h.at[pl.ds(base, BPW)])


def _tc_loss_body(sim_ref, out_ref):
    x = sim_ref[...]
    col = lax.broadcasted_iota(jnp.int32, x.shape, 1)
    ls = jax.nn.log_sigmoid(x)
    out_ref[0, 0] = -jnp.sum(jnp.where(col < NSAM, ls, 0.0)) / B


def kernel(cur, ctx, neg, W_in, W_out):
    cur = cur.astype(jnp.int32)
    ctx = ctx.astype(jnp.int32)
    neg = neg.astype(jnp.int32)

    win2 = W_in.reshape(V // 2, D2)
    wout2 = W_out.reshape(V // 2, D2)
    curi = cur >> 1
    curo = (cur & 1) * D
    ctxi = (ctx >> 1).reshape(B * NCTX)
    negi = (neg >> 1).reshape(B * NNEG)
    samu = jnp.concatenate(
        [ctx, neg, jnp.zeros((B, NG * L - NSAM), jnp.int32)], axis=1)

    sc_sim = functools.partial(
        pl.kernel,
        out_type=jax.ShapeDtypeStruct((B, NG * L), jnp.float32),
        mesh=plsc.VectorSubcoreMesh(core_axis_name="c", subcore_axis_name="s"),
        scratch_types=[
            pltpu.VMEM((BPW,), jnp.int32),
            pltpu.VMEM((BPW,), jnp.int32),
            pltpu.VMEM((BPW * NCTX,), jnp.int32),
            pltpu.VMEM((BPW * NNEG,), jnp.int32),
            pltpu.VMEM((BPW, NG * L), jnp.int32),
            pltpu.VMEM((BPW, D2), jnp.float32),
            pltpu.VMEM((2, CROWS, D2), jnp.float32),
            pltpu.VMEM((BPW, NG * L), jnp.float32),
            pltpu.SemaphoreType.DMA,
            pltpu.SemaphoreType.DMA,
        ],
        compiler_params=pltpu.CompilerParams(
            needs_layout_passes=False, use_tc_tiling_on_sc=True),
    )(_sc_sim_body)

    sim = sc_sim(curi, curo, ctxi, negi, samu, win2, wout2)

    loss = pl.pallas_call(
        _tc_loss_body,
        out_shape=jax.ShapeDtypeStruct((1, 1), jnp.float32),
        out_specs=pl.BlockSpec(memory_space=pltpu.SMEM),
    )(sim)
    return loss[0, 0]


# speedup vs baseline: 1.3295x; 1.3295x over previous
import functools

import jax
import jax.numpy as jnp
from jax import lax
from jax.experimental import pallas as pl
from jax.experimental.pallas import tpu as pltpu
from jax.experimental.pallas import tpu_sc as plsc

B, NCTX, NNEGS, V, D = 4096, 20, 5, 1000000, 64
NNEG = NNEGS * NCTX
NSAM = NCTX + NNEG
NG = 8
NW = 32
BPW = B // NW
L = 16
CB = 4
NCH = BPW // CB
CROWS = CB * NSAM


def _sc_sim_body(curv_h, ctx_h, neg_h, wout_h, out_h,
                 ctxi_v, negi_v, currow_v, rows_v, sim_v, sem0, sem1):
    wid = lax.axis_index("s") * 2 + lax.axis_index("c")
    base = wid * BPW
    sems = (sem0, sem1)

    pltpu.sync_copy(ctx_h.at[pl.ds(base * NCTX, BPW * NCTX)], ctxi_v)
    pltpu.sync_copy(neg_h.at[pl.ds(base * NNEG, BPW * NNEG)], negi_v)
    pltpu.sync_copy(curv_h.at[pl.ds(base, BPW)], currow_v)

    lanes = lax.iota(jnp.int32, L)
    rowsel0, rowstep = [], []
    for g in range(NG):
        sj = lanes + g * L
        row0 = jnp.where(sj < NCTX, sj, CB * NCTX + (sj - NCTX))
        step = jnp.where(sj < NCTX, NCTX, NNEG)
        valid = sj < NSAM
        rowsel0.append(jnp.where(valid, row0, 0))
        rowstep.append(jnp.where(valid, step, 0))
    sign = [jnp.where(lanes + g * L < NCTX, 1.0, -1.0).astype(jnp.float32)
            for g in range(NG)]
    zero = jnp.zeros((L,), jnp.float32)

    def issue(c, buf):
        pltpu.async_copy(wout_h.at[ctxi_v.at[pl.ds(c * CB * NCTX, CB * NCTX)]],
                         rows_v.at[buf, pl.ds(0, CB * NCTX)], sems[buf])
        pltpu.async_copy(wout_h.at[negi_v.at[pl.ds(c * CB * NNEG, CB * NNEG)]],
                         rows_v.at[buf, pl.ds(CB * NCTX, CB * NNEG)],
                         sems[buf])

    issue(0, 0)

    def body(cc, carry):
        for par in range(2):
            c = cc * 2 + par

            @pl.when(c + 1 < NCH)
            def _():
                issue(c + 1, 1 - par)

            pltpu.make_async_copy(
                wout_h.at[pl.ds(0, CROWS)],
                rows_v.at[par, pl.ds(0, CROWS)], sems[par]).wait()

            rows_c = rows_v.at[par]

            def bbody(bi, bcarry):
                b = c * CB + bi
                bvec = jnp.full((L,), b, jnp.int32)
                sel = [rowsel0[g] + bi * rowstep[g] for g in range(NG)]
                accs = [zero] * NG
                for d in range(D):
                    col = (lanes + d) & (D - 1)
                    crot = plsc.load_gather(currow_v, [bvec, col])
                    for g in range(NG):
                        accs[g] = accs[g] + plsc.load_gather(
                            rows_c, [sel[g], col]) * crot
                for g in range(NG):
                    sim_v[b, pl.ds(g * L, L)] = accs[g] * sign[g]
                return bcarry

            lax.fori_loop(0, CB, bbody, 0)
        return carry

    lax.fori_loop(0, NCH // 2, body, 0)
    pltpu.sync_copy(sim_v, out_h.at[pl.ds(base, BPW)])


def _tc_loss_body(sim_ref, out_ref):
    x = sim_ref[...]
    col = lax.broadcasted_iota(jnp.int32, x.shape, 1)
    ls = jax.nn.log_sigmoid(x)
    out_ref[0, 0] = -jnp.sum(jnp.where(col < NSAM, ls, 0.0)) / B


def kernel(cur, ctx, neg, W_in, W_out):
    cur = cur.astype(jnp.int32)
    ctx = ctx.astype(jnp.int32).reshape(B * NCTX)
    neg = neg.astype(jnp.int32).reshape(B * NNEG)
    curvec = jnp.take(W_in, cur, axis=0)

    sc_sim = functools.partial(
        pl.kernel,
        out_type=jax.ShapeDtypeStruct((B, NG * L), jnp.float32),
        mesh=plsc.VectorSubcoreMesh(core_axis_name="c", subcore_axis_name="s"),
        scratch_types=[
            pltpu.VMEM((BPW * NCTX,), jnp.int32),
            pltpu.VMEM((BPW * NNEG,), jnp.int32),
            pltpu.VMEM((BPW, D), jnp.float32),
            pltpu.VMEM((2, CROWS, D), jnp.float32),
            pltpu.VMEM((BPW, NG * L), jnp.float32),
            pltpu.SemaphoreType.DMA,
            pltpu.SemaphoreType.DMA,
        ],
        compiler_params=pltpu.CompilerParams(
            needs_layout_passes=False, use_tc_tiling_on_sc=False),
    )(_sc_sim_body)

    sim = sc_sim(curvec, ctx, neg, W_out)

    loss = pl.pallas_call(
        _tc_loss_body,
        out_shape=jax.ShapeDtypeStruct((1, 1), jnp.float32),
        out_specs=pl.BlockSpec(memory_space=pltpu.SMEM),
    )(sim)
    return loss[0, 0]


# speedup vs baseline: 1.3295x; 1.0000x over previous
import functools

import jax
import jax.numpy as jnp
from jax import lax
from jax.experimental import pallas as pl
from jax.experimental.pallas import tpu as pltpu
from jax.experimental.pallas import tpu_sc as plsc

B, NCTX, NNEGS, V, D = 4096, 20, 5, 1000000, 64
NNEG = NNEGS * NCTX
NSAM = NCTX + NNEG
NG = 8
NW = 32
BPW = B // NW
L = 16
CB = 4
NCH = BPW // CB
CROWS = CB * NSAM


def _sc_sim_body(wout_h, ctx_h, neg_h, curv_h, out_h,
                 ctxi_v, negi_v, currow_v, rows_v, sim_v, sem0, sem1):
    wid = lax.axis_index("s") * 2 + lax.axis_index("c")
    base = wid * BPW
    sems = (sem0, sem1)

    pltpu.sync_copy(ctx_h.at[pl.ds(base * NCTX, BPW * NCTX)], ctxi_v)
    pltpu.sync_copy(neg_h.at[pl.ds(base * NNEG, BPW * NNEG)], negi_v)
    pltpu.sync_copy(curv_h.at[pl.ds(base, BPW)], currow_v)

    lanes = lax.iota(jnp.int32, L)
    rowsel0, rowstep = [], []
    for g in range(NG):
        sj = lanes + g * L
        row0 = jnp.where(sj < NCTX, sj, CB * NCTX + (sj - NCTX))
        step = jnp.where(sj < NCTX, NCTX, NNEG)
        valid = sj < NSAM
        rowsel0.append(jnp.where(valid, row0, 0))
        rowstep.append(jnp.where(valid, step, 0))
    sign = [jnp.where(lanes + g * L < NCTX, 1.0, -1.0).astype(jnp.float32)
            for g in range(NG)]
    zero = jnp.zeros((L,), jnp.float32)

    def issue(c, buf):
        pltpu.async_copy(wout_h.at[ctxi_v.at[pl.ds(c * CB * NCTX, CB * NCTX)]],
                         rows_v.at[buf, pl.ds(0, CB * NCTX)], sems[buf])
        pltpu.async_copy(wout_h.at[negi_v.at[pl.ds(c * CB * NNEG, CB * NNEG)]],
                         rows_v.at[buf, pl.ds(CB * NCTX, CB * NNEG)],
                         sems[buf])

    issue(0, 0)

    def body(cc, carry):
        for par in range(2):
            c = cc * 2 + par

            @pl.when(c + 1 < NCH)
            def _():
                issue(c + 1, 1 - par)

            pltpu.make_async_copy(
                wout_h.at[pl.ds(0, CROWS)],
                rows_v.at[par, pl.ds(0, CROWS)], sems[par]).wait()

            rows_c = rows_v.at[par]

            def bbody(bi, bcarry):
                b = c * CB + bi
                bvec = jnp.full((L,), b, jnp.int32)
                sel = [rowsel0[g] + bi * rowstep[g] for g in range(NG)]
                accs = [zero] * NG
                for d in range(D):
                    col = (lanes + d) & (D - 1)
                    crot = plsc.load_gather(currow_v, [bvec, col])
                    for g in range(NG):
                        accs[g] = accs[g] + plsc.load_gather(
                            rows_c, [sel[g], col]) * crot
                for g in range(NG):
                    sim_v[b, pl.ds(g * L, L)] = accs[g] * sign[g]
                return bcarry

            lax.fori_loop(0, CB, bbody, 0)
        return carry

    lax.fori_loop(0, NCH // 2, body, 0)
    pltpu.sync_copy(sim_v, out_h.at[pl.ds(base, BPW)])


def _tc_loss_body(sim_ref, out_ref):
    x = sim_ref[...]
    col = lax.broadcasted_iota(jnp.int32, x.shape, 1)
    ls = jax.nn.log_sigmoid(x)
    out_ref[0, 0] = -jnp.sum(jnp.where(col < NSAM, ls, 0.0)) / B


def kernel(cur, ctx, neg, W_in, W_out):
    cur = cur.astype(jnp.int32)
    ctx = ctx.astype(jnp.int32).reshape(B * NCTX)
    neg = neg.astype(jnp.int32).reshape(B * NNEG)
    curvec = jnp.take(W_in, cur, axis=0)

    sc_sim = functools.partial(
        pl.kernel,
        out_type=jax.ShapeDtypeStruct((B, NG * L), jnp.float32),
        mesh=plsc.VectorSubcoreMesh(core_axis_name="c", subcore_axis_name="s"),
        scratch_types=[
            pltpu.VMEM((BPW * NCTX,), jnp.int32),
            pltpu.VMEM((BPW * NNEG,), jnp.int32),
            pltpu.VMEM((BPW, D), jnp.float32),
            pltpu.VMEM((2, CROWS, D), jnp.float32),
            pltpu.VMEM((BPW, NG * L), jnp.float32),
            pltpu.SemaphoreType.DMA,
            pltpu.SemaphoreType.DMA,
        ],
        compiler_params=pltpu.CompilerParams(
            needs_layout_passes=False, use_tc_tiling_on_sc=False),
    )(_sc_sim_body)

    sim = sc_sim(W_out, ctx, neg, curvec)

    loss = pl.pallas_call(
        _tc_loss_body,
        out_shape=jax.ShapeDtypeStruct((1, 1), jnp.float32),
        out_specs=pl.BlockSpec(memory_space=pltpu.SMEM),
    )(sim)
    return loss[0, 0]
